# baseline (speedup 1.0000x reference)
import jax
import jax.numpy as jnp
from jax import lax
from jax.experimental import pallas as pl
from jax.experimental.pallas import tpu as pltpu
from jax.experimental.pallas import tpu_sc as plsc

N = 10000
E = 320000
D = 128
H = 128
FC = 32

NC = 2
NS = 16
NW = NC * NS
CH = 128
NCH = 80
EPAD = NW * NCH * CH
NROW = EPAD // CH

SLAB = 640
NPDEG = NS * SLAB
NPA = 10016
HCH = NCH // 2

_MESH = plsc.VectorSubcoreMesh(
    core_axis_name="c", subcore_axis_name="s", num_cores=NC, num_subcores=NS
)


def _deg_body(dst2_hbm, out0_hbm, out1_hbm, di_v, ones_v, zer_v, acc_sh):
  c = lax.axis_index("c")
  s = lax.axis_index("s")
  crow = (c * NS + s) * NCH

  def fill(i, _):
    ones_v[pl.ds(i * 16, 16)] = jnp.full((16,), 1.0, jnp.float32)
    return 0

  lax.fori_loop(0, CH // 16, fill, 0)

  def fillz(i, _):
    zer_v[pl.ds(i * 16, 16)] = jnp.zeros((16,), jnp.float32)
    return 0

  lax.fori_loop(0, SLAB // 16, fillz, 0)

  pltpu.sync_copy(zer_v, acc_sh.at[pl.ds(s * SLAB, SLAB)])
  pltpu.sync_copy(dst2_hbm.at[pl.ds(crow, NCH)], di_v)

  plsc.subcore_barrier()

  def body(j, _):
    pltpu.sync_copy(ones_v, acc_sh.at[di_v.at[j]], add=True)
    return 0

  lax.fori_loop(0, NCH, body, 0)

  plsc.subcore_barrier()

  @pl.when(c == 0)
  def _():
    pltpu.sync_copy(acc_sh.at[pl.ds(s * SLAB, SLAB)],
                    out0_hbm.at[pl.ds(s * SLAB, SLAB)])

  @pl.when(c == 1)
  def _():
    pltpu.sync_copy(acc_sh.at[pl.ds(s * SLAB, SLAB)],
                    out1_hbm.at[pl.ds(s * SLAB, SLAB)])


_deg_call = pl.kernel(
    _deg_body,
    out_type=(jax.ShapeDtypeStruct((NPDEG,), jnp.float32),
              jax.ShapeDtypeStruct((NPDEG,), jnp.float32)),
    mesh=_MESH,
    scratch_types=[
        pltpu.VMEM((NCH, CH), jnp.int32),
        pltpu.VMEM((CH,), jnp.float32),
        pltpu.VMEM((SLAB,), jnp.float32),
        pltpu.VMEM_SHARED((NPDEG,), jnp.float32),
    ],
)


def _agg_body(t_hbm, src2_hbm, dst2_hbm, out0_hbm, out1_hbm,
              si_v, di_v, rows0, rows1, acc_sh, gs0, gs1):
  c = lax.axis_index("c")
  s = lax.axis_index("s")
  crow = (c * NS + s) * NCH
  row0 = s * SLAB
  last = s == NS - 1

  def fz(i, _):
    rows0[i // 8, pl.ds((i % 8) * 16, 16)] = jnp.zeros((16,), jnp.float32)
    return 0

  lax.fori_loop(0, CH * (D // 16), fz, 0)

  @pl.when(jnp.logical_not(last))
  def _():
    for k in range(SLAB // CH):
      pltpu.sync_copy(rows0, acc_sh.at[pl.ds(row0 + CH * k, CH)])

  @pl.when(last)
  def _():
    for k in range(3):
      pltpu.sync_copy(rows0, acc_sh.at[pl.ds(row0 + CH * k, CH)])
    pltpu.sync_copy(rows0.at[pl.ds(0, 32)],
                    acc_sh.at[pl.ds(row0 + 3 * CH, 32)])

  plsc.subcore_barrier()

  rows = (rows0, rows1)
  gs = (gs0, gs1)

  for phase in range(NCH // HCH):
    pltpu.sync_copy(src2_hbm.at[pl.ds(crow + phase * HCH, HCH)], si_v)
    pltpu.sync_copy(dst2_hbm.at[pl.ds(crow + phase * HCH, HCH)], di_v)

    pltpu.async_copy(t_hbm.at[si_v.at[0]], rows0, gs0).wait()

    def step(j, b):
      nb = 1 - b
      desc = pltpu.async_copy(t_hbm.at[si_v.at[j + 1]], rows[nb], gs[nb])
      pltpu.sync_copy(rows[b], acc_sh.at[di_v.at[j]], add=True)
      desc.wait()

    def outer(g, _):
      step(2 * g, 0)
      step(2 * g + 1, 1)
      return 0

    lax.fori_loop(0, HCH // 2 - 1, outer, 0)
    step(HCH - 2, 0)
    pltpu.sync_copy(rows1, acc_sh.at[di_v.at[HCH - 1]], add=True)

  plsc.subcore_barrier()

  def writeback(out_ref):
    @pl.when(jnp.logical_not(last))
    def _():
      for k in range(SLAB // CH):
        pltpu.sync_copy(acc_sh.at[pl.ds(row0 + CH * k, CH)],
                        out_ref.at[pl.ds(row0 + CH * k, CH)])

    @pl.when(last)
    def _():
      for k in range(3):
        pltpu.sync_copy(acc_sh.at[pl.ds(row0 + CH * k, CH)],
                        out_ref.at[pl.ds(row0 + CH * k, CH)])
      pltpu.sync_copy(acc_sh.at[pl.ds(row0 + 3 * CH, 16)],
                      out_ref.at[pl.ds(row0 + 3 * CH, 16)])

  @pl.when(c == 0)
  def _():
    writeback(out0_hbm)

  @pl.when(c == 1)
  def _():
    writeback(out1_hbm)


_agg_call = pl.kernel(
    _agg_body,
    out_type=(jax.ShapeDtypeStruct((NPA, D), jnp.float32),
              jax.ShapeDtypeStruct((NPA, D), jnp.float32)),
    mesh=_MESH,
    scratch_types=[
        pltpu.VMEM((HCH, CH), jnp.int32),
        pltpu.VMEM((HCH, CH), jnp.int32),
        pltpu.VMEM((CH, D), jnp.float32),
        pltpu.VMEM((CH, D), jnp.float32),
        pltpu.VMEM_SHARED((NPA, D), jnp.float32),
        pltpu.SemaphoreType.DMA,
        pltpu.SemaphoreType.DMA,
    ],
)


RB = 2000
_GRID = N // RB


def _rows(i):
  return (i, 0)


def _full(i):
  return (0, 0)


def _dis(d0_ref, d1_ref):
  return lax.rsqrt(1.0 + d0_ref[...] + d1_ref[...])


def _mm1_body(x_ref, w_ref, d0_ref, d1_ref, o_ref):
  o_ref[...] = jnp.dot(x_ref[...], w_ref[...],
                       preferred_element_type=jnp.float32) * _dis(d0_ref, d1_ref)


def _mm2_body(p0_ref, p1_ref, t_ref, d0_ref, d1_ref, b_ref, w_ref, o_ref):
  dis = _dis(d0_ref, d1_ref)
  h = jnp.maximum(dis * (p0_ref[...] + p1_ref[...] + t_ref[...]) + b_ref[...], 0.0)
  o_ref[...] = jnp.dot(h, w_ref[...], preferred_element_type=jnp.float32) * dis


def _head_body(p0_ref, p1_ref, t_ref, d0_ref, d1_ref, b_ref,
               wf_ref, bf_ref, wo_ref, bo_ref, o_ref):
  dis = _dis(d0_ref, d1_ref)
  h2 = jnp.maximum(dis * (p0_ref[...] + p1_ref[...] + t_ref[...]) + b_ref[...], 0.0)
  h3 = jnp.maximum(jnp.dot(h2, wf_ref[...], preferred_element_type=jnp.float32)
                   + bf_ref[...], 0.0)
  o_ref[...] = jnp.dot(h3, wo_ref[...], preferred_element_type=jnp.float32) + bo_ref[...]


def _row_spec(cols):
  return pl.BlockSpec((RB, cols), _rows)


def _w_spec(r, c):
  return pl.BlockSpec((r, c), _full)


_mm1 = pl.pallas_call(
    _mm1_body,
    grid=(_GRID,),
    in_specs=[_row_spec(D), _w_spec(D, H), _row_spec(1), _row_spec(1)],
    out_specs=_row_spec(H),
    out_shape=jax.ShapeDtypeStruct((N, H), jnp.float32),
)

_mm2 = pl.pallas_call(
    _mm2_body,
    grid=(_GRID,),
    in_specs=[_row_spec(H), _row_spec(H), _row_spec(H), _row_spec(1),
              _row_spec(1), _w_spec(1, H), _w_spec(H, H)],
    out_specs=_row_spec(H),
    out_shape=jax.ShapeDtypeStruct((N, H), jnp.float32),
)

_head = pl.pallas_call(
    _head_body,
    grid=(_GRID,),
    in_specs=[_row_spec(H), _row_spec(H), _row_spec(H), _row_spec(1),
              _row_spec(1), _w_spec(1, H), _w_spec(H, FC), _w_spec(1, FC),
              _w_spec(FC, 1), _w_spec(1, 1)],
    out_specs=_row_spec(1),
    out_shape=jax.ShapeDtypeStruct((N, 1), jnp.float32),
)


def kernel(x, edge_index, W1, b1, W2, b2, Wf, bf, Wo, bo):
  src = edge_index[0]
  dst = edge_index[1]
  pad = EPAD - E
  src2 = jnp.concatenate([src, jnp.zeros((pad,), jnp.int32)]).reshape(NROW, CH)
  dst2 = jnp.concatenate([dst, jnp.full((pad,), N, jnp.int32)]).reshape(NROW, CH)

  deg0, deg1 = _deg_call(dst2)
  d0 = deg0[:N, None]
  d1 = deg1[:N, None]

  t1 = _mm1(x, W1, d0, d1)
  p0, p1 = _agg_call(t1, src2, dst2)
  t2 = _mm2(p0, p1, t1, d0, d1, b1[None, :], W2)
  q0, q1 = _agg_call(t2, src2, dst2)
  return _head(q0, q1, t2, d0, d1, b2[None, :],
               Wf, bf[None, :], Wo, bo[None, :])


# speedup vs baseline: 3.3322x; 3.3322x over previous
import jax
import jax.numpy as jnp
from jax import lax
from jax.experimental import pallas as pl
from jax.experimental.pallas import tpu as pltpu
from jax.experimental.pallas import tpu_sc as plsc

N = 10000
E = 320000
D = 128
H = 128
FC = 32

NC = 2
NS = 16
NW = NC * NS
CH = 128
NCH = 80
EPAD = NW * NCH * CH
NROW = EPAD // CH

SLAB = 640
NPDEG = NS * SLAB
NPA = 10128
HCH = NCH // 2

_MESH = plsc.VectorSubcoreMesh(
    core_axis_name="c", subcore_axis_name="s", num_cores=NC, num_subcores=NS
)


def _deg_body(dst2_hbm, out0_hbm, out1_hbm, di_v, ones_v, zer_v, acc_sh):
  c = lax.axis_index("c")
  s = lax.axis_index("s")
  crow = (c * NS + s) * NCH

  def fill(i, _):
    ones_v[pl.ds(i * 16, 16)] = jnp.full((16,), 1.0, jnp.float32)
    return 0

  lax.fori_loop(0, CH // 16, fill, 0)

  def fillz(i, _):
    zer_v[pl.ds(i * 16, 16)] = jnp.zeros((16,), jnp.float32)
    return 0

  lax.fori_loop(0, SLAB // 16, fillz, 0)

  pltpu.sync_copy(zer_v, acc_sh.at[pl.ds(s * SLAB, SLAB)])
  pltpu.sync_copy(dst2_hbm.at[pl.ds(crow, NCH)], di_v)

  plsc.subcore_barrier()

  def body(j, _):
    pltpu.sync_copy(ones_v, acc_sh.at[di_v.at[j]], add=True)
    return 0

  lax.fori_loop(0, NCH, body, 0)

  plsc.subcore_barrier()

  @pl.when(c == 0)
  def _():
    pltpu.sync_copy(acc_sh.at[pl.ds(s * SLAB, SLAB)],
                    out0_hbm.at[pl.ds(s * SLAB, SLAB)])

  @pl.when(c == 1)
  def _():
    pltpu.sync_copy(acc_sh.at[pl.ds(s * SLAB, SLAB)],
                    out1_hbm.at[pl.ds(s * SLAB, SLAB)])


_deg_call = pl.kernel(
    _deg_body,
    out_type=(jax.ShapeDtypeStruct((NPDEG,), jnp.float32),
              jax.ShapeDtypeStruct((NPDEG,), jnp.float32)),
    mesh=_MESH,
    scratch_types=[
        pltpu.VMEM((NCH, CH), jnp.int32),
        pltpu.VMEM((CH,), jnp.float32),
        pltpu.VMEM((SLAB,), jnp.float32),
        pltpu.VMEM_SHARED((NPDEG,), jnp.float32),
    ],
)


def _agg_body(t_hbm, src2_hbm, dst2_hbm, out0_hbm, out1_hbm,
              si_v, di_v, rows0, rows1, acc_sh, gs0, gs1):
  c = lax.axis_index("c")
  s = lax.axis_index("s")
  crow = (c * NS + s) * NCH
  row0 = s * SLAB
  last = s == NS - 1

  def fz(i, _):
    rows0[i // 8, pl.ds((i % 8) * 16, 16)] = jnp.zeros((16,), jnp.float32)
    return 0

  lax.fori_loop(0, CH * (D // 16), fz, 0)

  @pl.when(jnp.logical_not(last))
  def _():
    for k in range(SLAB // CH):
      pltpu.sync_copy(rows0, acc_sh.at[pl.ds(row0 + CH * k, CH)])

  @pl.when(last)
  def _():
    for k in range(3):
      pltpu.sync_copy(rows0, acc_sh.at[pl.ds(row0 + CH * k, CH)])
    pltpu.sync_copy(rows0.at[pl.ds(0, 32)],
                    acc_sh.at[pl.ds(row0 + 3 * CH, 32)])

  plsc.subcore_barrier()

  rows = (rows0, rows1)
  gs = (gs0, gs1)

  for phase in range(NCH // HCH):
    pltpu.sync_copy(src2_hbm.at[pl.ds(crow + phase * HCH, HCH)], si_v)
    pltpu.sync_copy(dst2_hbm.at[pl.ds(crow + phase * HCH, HCH)], di_v)

    pltpu.async_copy(t_hbm.at[si_v.at[0]], rows0, gs0).wait()

    def step(j, b):
      nb = 1 - b
      desc = pltpu.async_copy(t_hbm.at[si_v.at[j + 1]], rows[nb], gs[nb])
      pltpu.sync_copy(rows[b], acc_sh.at[di_v.at[j]], add=True)
      desc.wait()

    def outer(g, _):
      step(2 * g, 0)
      step(2 * g + 1, 1)
      return 0

    lax.fori_loop(0, HCH // 2 - 1, outer, 0)
    step(HCH - 2, 0)
    pltpu.sync_copy(rows1, acc_sh.at[di_v.at[HCH - 1]], add=True)

  plsc.subcore_barrier()

  def writeback(out_ref):
    @pl.when(jnp.logical_not(last))
    def _():
      for k in range(SLAB // CH):
        pltpu.sync_copy(acc_sh.at[pl.ds(row0 + CH * k, CH)],
                        out_ref.at[pl.ds(row0 + CH * k, CH)])

    @pl.when(last)
    def _():
      for k in range(3):
        pltpu.sync_copy(acc_sh.at[pl.ds(row0 + CH * k, CH)],
                        out_ref.at[pl.ds(row0 + CH * k, CH)])
      pltpu.sync_copy(acc_sh.at[pl.ds(row0 + 3 * CH, 16)],
                      out_ref.at[pl.ds(row0 + 3 * CH, 16)])

  @pl.when(c == 0)
  def _():
    writeback(out0_hbm)

  @pl.when(c == 1)
  def _():
    writeback(out1_hbm)


_agg_call = pl.kernel(
    _agg_body,
    out_type=(jax.ShapeDtypeStruct((NPA, D), jnp.float32),
              jax.ShapeDtypeStruct((NPA, D), jnp.float32)),
    mesh=_MESH,
    scratch_types=[
        pltpu.VMEM((HCH, CH), jnp.int32),
        pltpu.VMEM((HCH, CH), jnp.int32),
        pltpu.VMEM((CH, D), jnp.float32),
        pltpu.VMEM((CH, D), jnp.float32),
        pltpu.VMEM_SHARED((NPA, D), jnp.float32),
        pltpu.SemaphoreType.DMA,
        pltpu.SemaphoreType.DMA,
    ],
)


RB = 2000
_GRID = N // RB


def _rows(i):
  return (i, 0)


def _full(i):
  return (0, 0)


def _dis(d0_ref, d1_ref):
  return lax.rsqrt(1.0 + d0_ref[...] + d1_ref[...])


def _mm1_body(x_ref, w_ref, d0_ref, d1_ref, o_ref):
  o_ref[...] = jnp.dot(x_ref[...], w_ref[...],
                       preferred_element_type=jnp.float32) * _dis(d0_ref, d1_ref)


def _mm2_body(p0_ref, p1_ref, t_ref, d0_ref, d1_ref, b_ref, w_ref, o_ref):
  dis = _dis(d0_ref, d1_ref)
  h = jnp.maximum(dis * (p0_ref[...] + p1_ref[...] + t_ref[...]) + b_ref[...], 0.0)
  o_ref[...] = jnp.dot(h, w_ref[...], preferred_element_type=jnp.float32) * dis


def _head_body(p0_ref, p1_ref, t_ref, d0_ref, d1_ref, b_ref,
               wf_ref, bf_ref, wo_ref, bo_ref, o_ref):
  dis = _dis(d0_ref, d1_ref)
  h2 = jnp.maximum(dis * (p0_ref[...] + p1_ref[...] + t_ref[...]) + b_ref[...], 0.0)
  h3 = jnp.maximum(jnp.dot(h2, wf_ref[...], preferred_element_type=jnp.float32)
                   + bf_ref[...], 0.0)
  o_ref[...] = jnp.dot(h3, wo_ref[...], preferred_element_type=jnp.float32) + bo_ref[...]


def _row_spec(cols):
  return pl.BlockSpec((RB, cols), _rows)


def _w_spec(r, c):
  return pl.BlockSpec((r, c), _full)


_mm1 = pl.pallas_call(
    _mm1_body,
    grid=(_GRID,),
    in_specs=[_row_spec(D), _w_spec(D, H), _row_spec(1), _row_spec(1)],
    out_specs=_row_spec(H),
    out_shape=jax.ShapeDtypeStruct((N, H), jnp.float32),
)

_mm2 = pl.pallas_call(
    _mm2_body,
    grid=(_GRID,),
    in_specs=[_row_spec(H), _row_spec(H), _row_spec(H), _row_spec(1),
              _row_spec(1), _w_spec(1, H), _w_spec(H, H)],
    out_specs=_row_spec(H),
    out_shape=jax.ShapeDtypeStruct((N, H), jnp.float32),
)

_head = pl.pallas_call(
    _head_body,
    grid=(_GRID,),
    in_specs=[_row_spec(H), _row_spec(H), _row_spec(H), _row_spec(1),
              _row_spec(1), _w_spec(1, H), _w_spec(H, FC), _w_spec(1, FC),
              _w_spec(FC, 1), _w_spec(1, 1)],
    out_specs=_row_spec(1),
    out_shape=jax.ShapeDtypeStruct((N, 1), jnp.float32),
)


def kernel(x, edge_index, W1, b1, W2, b2, Wf, bf, Wo, bo):
  src = edge_index[0]
  dst = edge_index[1]
  pad = EPAD - E
  pad_i = jnp.arange(pad, dtype=jnp.int32) % 128
  src2 = jnp.concatenate([src, pad_i]).reshape(NROW, CH)
  dst2 = jnp.concatenate([dst, N + pad_i]).reshape(NROW, CH)

  deg0, deg1 = _deg_call(dst2)
  d0 = deg0[:N, None]
  d1 = deg1[:N, None]

  t1 = _mm1(x, W1, d0, d1)
  p0, p1 = _agg_call(t1, src2, dst2)
  t2 = _mm2(p0, p1, t1, d0, d1, b1[None, :], W2)
  q0, q1 = _agg_call(t2, src2, dst2)
  return _head(q0, q1, t2, d0, d1, b2[None, :],
               Wf, bf[None, :], Wo, bo[None, :])
